# Initial kernel scaffold; baseline (speedup 1.0000x reference)
import functools

import jax
import jax.numpy as jnp
from jax import lax
from jax.experimental import pallas as pl
from jax.experimental.pallas import tpu as pltpu
from jax.experimental.pallas import tpu_sc as plsc

NUM_FRAMES = 4096
NUM_AUDIO = 8192
AUDIO_DIM = 1280
VIS_DIM = 512
OUT_DIM = VIS_DIM + AUDIO_DIM

FB = 64
NB = NUM_FRAMES // FB

NC = 2
NS = 16
NW = NC * NS
B_PER_W = NUM_FRAMES // NW
CHUNK = 32


def _argmin_body(f_ref, a_ref, o_ref):
    f = f_ref[...]
    a = a_ref[...]
    d = jnp.abs(a - f)
    m = jnp.min(d, axis=1, keepdims=True)
    ii = lax.broadcasted_iota(jnp.int32, d.shape, 1)
    cand = jnp.where(d == m, ii, NUM_AUDIO)
    o_ref[0, 0, :] = jnp.min(cand, axis=1)


def _closest_idx(frame_ts, audio_ts):
    f2 = frame_ts.reshape(NUM_FRAMES, 1)
    a2 = audio_ts.reshape(1, NUM_AUDIO)
    idx3 = pl.pallas_call(
        _argmin_body,
        grid=(NB,),
        in_specs=[
            pl.BlockSpec((FB, 1), lambda i: (i, 0)),
            pl.BlockSpec((1, NUM_AUDIO), lambda i: (0, 0)),
        ],
        out_specs=pl.BlockSpec((1, 1, FB), lambda i: (i, 0, 0)),
        out_shape=jax.ShapeDtypeStruct((NB, 1, FB), jnp.int32),
    )(f2, a2)
    return idx3.reshape(NUM_FRAMES)


def _sc_gather_body(vis_hbm, audio_hbm, idx_hbm, out_hbm, idx_v, rows_v, vis_v, sem):
    wid = lax.axis_index("s") * NC + lax.axis_index("c")
    base = wid * B_PER_W
    pltpu.sync_copy(idx_hbm.at[pl.ds(base, B_PER_W)], idx_v)

    @pl.loop(0, B_PER_W // CHUNK)
    def _(c):
        off = base + c * CHUNK
        pltpu.async_copy(audio_hbm.at[idx_v.at[pl.ds(c * CHUNK, CHUNK)]], rows_v, sem).wait()
        pltpu.sync_copy(rows_v, out_hbm.at[pl.ds(off, CHUNK), pl.ds(VIS_DIM, AUDIO_DIM)])
        pltpu.sync_copy(vis_hbm.at[pl.ds(off, CHUNK)], vis_v)
        pltpu.sync_copy(vis_v, out_hbm.at[pl.ds(off, CHUNK), pl.ds(0, VIS_DIM)])


@functools.partial(
    pl.kernel,
    out_type=jax.ShapeDtypeStruct((NUM_FRAMES, OUT_DIM), jnp.float32),
    mesh=plsc.VectorSubcoreMesh(core_axis_name="c", subcore_axis_name="s"),
    scratch_types=[
        pltpu.VMEM((B_PER_W,), jnp.int32),
        pltpu.VMEM((CHUNK, AUDIO_DIM), jnp.float32),
        pltpu.VMEM((CHUNK, VIS_DIM), jnp.float32),
        pltpu.SemaphoreType.DMA,
    ],
)
def _sc_gather(vis_hbm, audio_hbm, idx_hbm, out_hbm, idx_v, rows_v, vis_v, sem):
    _sc_gather_body(vis_hbm, audio_hbm, idx_hbm, out_hbm, idx_v, rows_v, vis_v, sem)


def kernel(visual_features, audio_features, audio_timestamps, frame_timestamps):
    idx = _closest_idx(frame_timestamps, audio_timestamps)
    return _sc_gather(visual_features, audio_features, idx)


# speedup vs baseline: 1.3597x; 1.3597x over previous
import functools

import jax
import jax.numpy as jnp
from jax import lax
from jax.experimental import pallas as pl
from jax.experimental.pallas import tpu as pltpu
from jax.experimental.pallas import tpu_sc as plsc

NUM_FRAMES = 4096
NUM_AUDIO = 8192
AUDIO_DIM = 1280
VIS_DIM = 512
OUT_DIM = VIS_DIM + AUDIO_DIM

FB = 64
NB = NUM_FRAMES // FB

NC = 2
NS = 16
NW = NC * NS
B_PER_W = NUM_FRAMES // NW
CHUNK = 32


def _argmin_body(f_ref, a_ref, o_ref):
    f = f_ref[...]
    a = a_ref[...]
    d = jnp.abs(a - f)
    m = jnp.min(d, axis=1, keepdims=True)
    ii = lax.broadcasted_iota(jnp.int32, d.shape, 1)
    cand = jnp.where(d == m, ii, NUM_AUDIO)
    o_ref[0, 0, :] = jnp.min(cand, axis=1)


def _closest_idx(frame_ts, audio_ts):
    f2 = frame_ts.reshape(NUM_FRAMES, 1)
    a2 = audio_ts.reshape(1, NUM_AUDIO)
    idx3 = pl.pallas_call(
        _argmin_body,
        grid=(NB,),
        in_specs=[
            pl.BlockSpec((FB, 1), lambda i: (i, 0)),
            pl.BlockSpec((1, NUM_AUDIO), lambda i: (0, 0)),
        ],
        out_specs=pl.BlockSpec((1, 1, FB), lambda i: (i, 0, 0)),
        out_shape=jax.ShapeDtypeStruct((NB, 1, FB), jnp.int32),
    )(f2, a2)
    return idx3.reshape(NUM_FRAMES)


def _sc_gather_body(vis_hbm, audio_hbm, idx_hbm, out_hbm, idx_v, rows_v, vis_v, sem):
    wid = lax.axis_index("s") * NC + lax.axis_index("c")
    base = wid * B_PER_W
    pltpu.sync_copy(idx_hbm.at[pl.ds(base, B_PER_W)], idx_v)

    @pl.loop(0, B_PER_W // CHUNK)
    def _(c):
        off = base + c * CHUNK
        pltpu.async_copy(audio_hbm.at[idx_v.at[pl.ds(c * CHUNK, CHUNK)]], rows_v, sem).wait()
        pltpu.sync_copy(rows_v, out_hbm.at[pl.ds(off, CHUNK), pl.ds(VIS_DIM, AUDIO_DIM)])
        pltpu.sync_copy(vis_hbm.at[pl.ds(off, CHUNK)], vis_v)
        pltpu.sync_copy(vis_v, out_hbm.at[pl.ds(off, CHUNK), pl.ds(0, VIS_DIM)])


@functools.lru_cache(maxsize=1)
def _build_sc_gather():
    return pl.kernel(
        _sc_gather_body,
        out_type=jax.ShapeDtypeStruct((NUM_FRAMES, OUT_DIM), jnp.float32),
        mesh=plsc.VectorSubcoreMesh(
            core_axis_name="c", subcore_axis_name="s", num_cores=NC, num_subcores=NS
        ),
        scratch_types=[
            pltpu.VMEM((B_PER_W,), jnp.int32),
            pltpu.VMEM((CHUNK, AUDIO_DIM), jnp.float32),
            pltpu.VMEM((CHUNK, VIS_DIM), jnp.float32),
            pltpu.SemaphoreType.DMA,
        ],
    )


def kernel(visual_features, audio_features, audio_timestamps, frame_timestamps):
    idx = _closest_idx(frame_timestamps, audio_timestamps)
    return _build_sc_gather()(visual_features, audio_features, idx)


# speedup vs baseline: 1.3877x; 1.0206x over previous
import functools

import jax
import jax.numpy as jnp
from jax import lax
from jax.experimental import pallas as pl
from jax.experimental.pallas import tpu as pltpu
from jax.experimental.pallas import tpu_sc as plsc

NUM_FRAMES = 4096
NUM_AUDIO = 8192
AUDIO_DIM = 1280
VIS_DIM = 512
OUT_DIM = VIS_DIM + AUDIO_DIM

FB = 64
NB = NUM_FRAMES // FB
CW = 128
NCH = NUM_AUDIO // CW

NC = 2
NS = 16
NW = NC * NS
B_PER_W = NUM_FRAMES // NW
CHUNK = 32


def _argmin_body(f_ref, a_ref, o_ref):
    f = f_ref[...]
    fb = jnp.broadcast_to(f, (FB, CW))

    def step(c, carry):
        mv, mc = carry
        a_row = a_ref[pl.ds(c, 1), :]
        d = jnp.abs(a_row - fb)
        lt = d < mv
        cf = c.astype(jnp.float32)
        return jnp.where(lt, d, mv), jnp.where(lt, cf, mc)

    mv0 = jnp.full((FB, CW), jnp.inf, jnp.float32)
    mc0 = jnp.zeros((FB, CW), jnp.float32)
    mv, mc = lax.fori_loop(0, NCH, step, (mv0, mc0), unroll=2)

    gmin = jnp.min(mv, axis=1, keepdims=True)
    lane = lax.broadcasted_iota(jnp.int32, (FB, CW), 1).astype(jnp.float32)
    gidx = mc * CW + lane
    cand = jnp.where(mv == gmin, gidx, jnp.float32(NUM_AUDIO))
    o_ref[0, 0, :] = jnp.min(cand, axis=1).astype(jnp.int32)


def _closest_idx(frame_ts, audio_ts):
    f2 = frame_ts.reshape(NUM_FRAMES, 1)
    a2 = audio_ts.reshape(NCH, CW)
    idx3 = pl.pallas_call(
        _argmin_body,
        grid=(NB,),
        in_specs=[
            pl.BlockSpec((FB, 1), lambda i: (i, 0)),
            pl.BlockSpec((NCH, CW), lambda i: (0, 0)),
        ],
        out_specs=pl.BlockSpec((1, 1, FB), lambda i: (i, 0, 0)),
        out_shape=jax.ShapeDtypeStruct((NB, 1, FB), jnp.int32),
    )(f2, a2)
    return idx3.reshape(NUM_FRAMES)


def _sc_gather_body(vis_hbm, audio_hbm, idx_hbm, out_hbm, idx_v, rows_v, vis_v, sem):
    wid = lax.axis_index("s") * NC + lax.axis_index("c")
    base = wid * B_PER_W
    pltpu.sync_copy(idx_hbm.at[pl.ds(base, B_PER_W)], idx_v)

    @pl.loop(0, B_PER_W // CHUNK)
    def _(c):
        off = base + c * CHUNK
        pltpu.async_copy(audio_hbm.at[idx_v.at[pl.ds(c * CHUNK, CHUNK)]], rows_v, sem).wait()
        pltpu.sync_copy(rows_v, out_hbm.at[pl.ds(off, CHUNK), pl.ds(VIS_DIM, AUDIO_DIM)])
        pltpu.sync_copy(vis_hbm.at[pl.ds(off, CHUNK)], vis_v)
        pltpu.sync_copy(vis_v, out_hbm.at[pl.ds(off, CHUNK), pl.ds(0, VIS_DIM)])


@functools.lru_cache(maxsize=1)
def _build_sc_gather():
    return pl.kernel(
        _sc_gather_body,
        out_type=jax.ShapeDtypeStruct((NUM_FRAMES, OUT_DIM), jnp.float32),
        mesh=plsc.VectorSubcoreMesh(
            core_axis_name="c", subcore_axis_name="s", num_cores=NC, num_subcores=NS
        ),
        scratch_types=[
            pltpu.VMEM((B_PER_W,), jnp.int32),
            pltpu.VMEM((CHUNK, AUDIO_DIM), jnp.float32),
            pltpu.VMEM((CHUNK, VIS_DIM), jnp.float32),
            pltpu.SemaphoreType.DMA,
        ],
    )


def kernel(visual_features, audio_features, audio_timestamps, frame_timestamps):
    idx = _closest_idx(frame_timestamps, audio_timestamps)
    return _build_sc_gather()(visual_features, audio_features, idx)


# speedup vs baseline: 1.4581x; 1.0507x over previous
import functools

import jax
import jax.numpy as jnp
from jax import lax
from jax.experimental import pallas as pl
from jax.experimental.pallas import tpu as pltpu
from jax.experimental.pallas import tpu_sc as plsc

NUM_FRAMES = 4096
NUM_AUDIO = 8192
AUDIO_DIM = 1280
VIS_DIM = 512
OUT_DIM = VIS_DIM + AUDIO_DIM

FB = 64
NB = NUM_FRAMES // FB
CW = 128
NCH = NUM_AUDIO // CW

NC = 2
NS = 16
NW = NC * NS
B_PER_W = NUM_FRAMES // NW
CHUNK = 32


def _argmin_body(f_ref, a_ref, o_ref):
    f = f_ref[...]
    fb = jnp.broadcast_to(f, (FB, CW))

    mv = jnp.abs(a_ref[0:1, :] - fb)
    mc = jnp.zeros((FB, CW), jnp.float32)
    for c in range(1, NCH):
        d = jnp.abs(a_ref[c:c + 1, :] - fb)
        lt = d < mv
        mv = jnp.where(lt, d, mv)
        mc = jnp.where(lt, jnp.float32(c), mc)

    gmin = jnp.min(mv, axis=1, keepdims=True)
    lane = lax.broadcasted_iota(jnp.int32, (FB, CW), 1).astype(jnp.float32)
    gidx = mc * CW + lane
    cand = jnp.where(mv == gmin, gidx, jnp.float32(NUM_AUDIO))
    o_ref[0, 0, :] = jnp.min(cand, axis=1).astype(jnp.int32)


def _closest_idx(frame_ts, audio_ts):
    f2 = frame_ts.reshape(NUM_FRAMES, 1)
    a2 = audio_ts.reshape(NCH, CW)
    idx3 = pl.pallas_call(
        _argmin_body,
        grid=(NB,),
        in_specs=[
            pl.BlockSpec((FB, 1), lambda i: (i, 0)),
            pl.BlockSpec((NCH, CW), lambda i: (0, 0)),
        ],
        out_specs=pl.BlockSpec((1, 1, FB), lambda i: (i, 0, 0)),
        out_shape=jax.ShapeDtypeStruct((NB, 1, FB), jnp.int32),
    )(f2, a2)
    return idx3.reshape(NUM_FRAMES)


def _sc_gather_body(vis_hbm, audio_hbm, idx_hbm, out_hbm, idx_v, rows_v, vis_v, sem):
    wid = lax.axis_index("s") * NC + lax.axis_index("c")
    base = wid * B_PER_W
    pltpu.sync_copy(idx_hbm.at[pl.ds(base, B_PER_W)], idx_v)

    @pl.loop(0, B_PER_W // CHUNK)
    def _(c):
        off = base + c * CHUNK
        pltpu.async_copy(audio_hbm.at[idx_v.at[pl.ds(c * CHUNK, CHUNK)]], rows_v, sem).wait()
        pltpu.sync_copy(rows_v, out_hbm.at[pl.ds(off, CHUNK), pl.ds(VIS_DIM, AUDIO_DIM)])
        pltpu.sync_copy(vis_hbm.at[pl.ds(off, CHUNK)], vis_v)
        pltpu.sync_copy(vis_v, out_hbm.at[pl.ds(off, CHUNK), pl.ds(0, VIS_DIM)])


@functools.lru_cache(maxsize=1)
def _build_sc_gather():
    return pl.kernel(
        _sc_gather_body,
        out_type=jax.ShapeDtypeStruct((NUM_FRAMES, OUT_DIM), jnp.float32),
        mesh=plsc.VectorSubcoreMesh(
            core_axis_name="c", subcore_axis_name="s", num_cores=NC, num_subcores=NS
        ),
        scratch_types=[
            pltpu.VMEM((B_PER_W,), jnp.int32),
            pltpu.VMEM((CHUNK, AUDIO_DIM), jnp.float32),
            pltpu.VMEM((CHUNK, VIS_DIM), jnp.float32),
            pltpu.SemaphoreType.DMA,
        ],
    )


def kernel(visual_features, audio_features, audio_timestamps, frame_timestamps):
    idx = _closest_idx(frame_timestamps, audio_timestamps)
    return _build_sc_gather()(visual_features, audio_features, idx)
